# baseline (speedup 1.0000x reference)
import functools

import jax
import jax.numpy as jnp
from jax import lax
from jax.experimental import pallas as pl
from jax.experimental.pallas import tpu as pltpu
from jax.experimental.pallas import tpu_sc as plsc

_TM = 512


def _dist_argmax_body(x_ref, et2_ref, dist_ref, idx_ref):
    x = x_ref[...]
    et2 = et2_ref[...]
    tm = x.shape[0]
    kk = et2.shape[1]

    acc = lax.dot_general(x, et2, (((1,), (0,)), ((), ())),
                          preferred_element_type=jnp.float32)
    x2 = jnp.sum(x * x, axis=1, keepdims=True)
    e2 = 0.25 * jnp.sum(et2 * et2, axis=0, keepdims=True)
    dist = acc - x2 - e2
    dist_ref[...] = dist

    lmax = jnp.max(dist, axis=1, keepdims=True)
    ii = lax.broadcasted_iota(jnp.int32, (1, kk), 1)
    keys = lax.bitcast_convert_type(ii | jnp.int32(0x3F800000), jnp.float32)
    masked = jnp.where(dist == lmax, keys, jnp.float32(2.0))
    kmin = jnp.min(masked, axis=1, keepdims=True)
    lidx = (lax.bitcast_convert_type(kmin, jnp.int32)
            & jnp.int32(0x007FFFFF))
    idx_ref[...] = lidx.reshape((tm,))


def _dist_argmax(xf, et):
    m, d = xf.shape
    kk = et.shape[1]
    return pl.pallas_call(
        _dist_argmax_body,
        grid=(m // _TM,),
        in_specs=[
            pl.BlockSpec((_TM, d), lambda i: (i, 0)),
            pl.BlockSpec((d, kk), lambda i: (0, 0)),
        ],
        out_specs=[
            pl.BlockSpec((_TM, kk), lambda i: (i, 0)),
            pl.BlockSpec((_TM,), lambda i: (i,)),
        ],
        out_shape=[
            jax.ShapeDtypeStruct((m, kk), jnp.float32),
            jax.ShapeDtypeStruct((m,), jnp.int32),
        ],
        compiler_params=pltpu.CompilerParams(
            dimension_semantics=("arbitrary",)),
    )(xf, et)


def _make_sc_gather(n_rows, d):
    info = plsc.get_sparse_core_info()
    nc, ns = info.num_cores, info.num_subcores
    nw = nc * ns
    b_per_w = n_rows // nw
    n_chunks = (b_per_w + 127) // 128
    chunk = b_per_w // n_chunks
    mesh = plsc.VectorSubcoreMesh(core_axis_name="c", subcore_axis_name="s")

    @functools.partial(
        pl.kernel, mesh=mesh,
        out_type=jax.ShapeDtypeStruct((n_rows, d), jnp.float32),
        scratch_types=[
            pltpu.VMEM((n_chunks, chunk), jnp.int32),
            pltpu.VMEM((n_chunks, chunk, d), jnp.float32),
            pltpu.SemaphoreType.DMA,
            pltpu.SemaphoreType.DMA,
        ],
    )
    def gather_k(idx_hbm, table_hbm, out_hbm, idx_v, rows_v, sem, sem2):
        wid = lax.axis_index("s") * nc + lax.axis_index("c")
        base = wid * b_per_w
        for c in range(n_chunks):
            pltpu.sync_copy(idx_hbm.at[pl.ds(base + c * chunk, chunk)],
                            idx_v.at[c])
        gathers = [
            pltpu.async_copy(table_hbm.at[idx_v.at[c]], rows_v.at[c], sem)
            for c in range(n_chunks)
        ]
        outs = []
        for c in range(n_chunks):
            gathers[c].wait()
            outs.append(pltpu.async_copy(
                rows_v.at[c], out_hbm.at[pl.ds(base + c * chunk, chunk)],
                sem2))
        for cp in outs:
            cp.wait()

    return gather_k


def kernel(x, inited, cluster_size, embed, embed_avg):
    b, s, d = x.shape
    kk = embed.shape[0]
    xf = x.reshape(-1, d)
    dist, idx = _dist_argmax(xf, (embed + embed).T)
    quantize = _make_sc_gather(xf.shape[0], d)(idx, embed)
    return (quantize.reshape(b, s, d), idx.reshape(b, s),
            dist.reshape(b, s, kk))


# speedup vs baseline: 1.0066x; 1.0066x over previous
import functools

import jax
import jax.numpy as jnp
from jax import lax
from jax.experimental import pallas as pl
from jax.experimental.pallas import tpu as pltpu
from jax.experimental.pallas import tpu_sc as plsc

_TM = 512


def _dist_argmax_body(x_ref, et2_ref, dist_ref, idx_ref):
    x = x_ref[...]
    et2 = et2_ref[...]
    tm = x.shape[0]
    kk = et2.shape[1]

    acc = lax.dot_general(x, et2, (((1,), (0,)), ((), ())),
                          preferred_element_type=jnp.float32)
    x2 = jnp.sum(x * x, axis=1, keepdims=True)
    e2 = 0.25 * jnp.sum(et2 * et2, axis=0, keepdims=True)
    dist = acc - x2 - e2
    dist_ref[...] = dist

    lmax = jnp.max(dist, axis=1, keepdims=True)
    ii = lax.broadcasted_iota(jnp.int32, (1, kk), 1)
    keys = lax.bitcast_convert_type(ii | jnp.int32(0x3F800000), jnp.float32)
    masked = jnp.where(dist == lmax, keys, jnp.float32(2.0))
    kmin = jnp.min(masked, axis=1, keepdims=True)
    lidx = (lax.bitcast_convert_type(kmin, jnp.int32)
            & jnp.int32(0x007FFFFF))
    idx_ref[...] = lidx.reshape((tm,))


def _dist_argmax(xf, et):
    m, d = xf.shape
    kk = et.shape[1]
    return pl.pallas_call(
        _dist_argmax_body,
        grid=(m // _TM,),
        in_specs=[
            pl.BlockSpec((_TM, d), lambda i: (i, 0)),
            pl.BlockSpec((d, kk), lambda i: (0, 0)),
        ],
        out_specs=[
            pl.BlockSpec((_TM, kk), lambda i: (i, 0)),
            pl.BlockSpec((_TM,), lambda i: (i,)),
        ],
        out_shape=[
            jax.ShapeDtypeStruct((m, kk), jnp.float32),
            jax.ShapeDtypeStruct((m,), jnp.int32),
        ],
        compiler_params=pltpu.CompilerParams(
            dimension_semantics=("arbitrary",)),
    )(xf, et)


def _make_sc_gather(n_rows, d):
    info = plsc.get_sparse_core_info()
    nc, ns = info.num_cores, info.num_subcores
    nw = nc * ns
    b_per_w = n_rows // nw
    n_chunks = (b_per_w + 127) // 128
    chunk = b_per_w // n_chunks
    mesh = plsc.VectorSubcoreMesh(core_axis_name="c", subcore_axis_name="s")

    @functools.partial(
        pl.kernel, mesh=mesh,
        out_type=jax.ShapeDtypeStruct((n_rows, d), jnp.float32),
        scratch_types=[
            pltpu.VMEM((n_chunks, chunk), jnp.int32),
            pltpu.VMEM((n_chunks, chunk, d), jnp.float32),
            pltpu.SemaphoreType.DMA,
        ],
    )
    def gather_k(idx_hbm, table_hbm, out_hbm, idx_v, rows_v, sem):
        wid = lax.axis_index("s") * nc + lax.axis_index("c")
        base = wid * b_per_w
        for c in range(n_chunks):
            pltpu.sync_copy(idx_hbm.at[pl.ds(base + c * chunk, chunk)],
                            idx_v.at[c])
        copies = [
            pltpu.async_copy(table_hbm.at[idx_v.at[c]], rows_v.at[c], sem)
            for c in range(n_chunks)
        ]
        for cp in copies:
            cp.wait()
        for c in range(n_chunks):
            pltpu.sync_copy(rows_v.at[c],
                            out_hbm.at[pl.ds(base + c * chunk, chunk)])

    return gather_k


def kernel(x, inited, cluster_size, embed, embed_avg):
    b, s, d = x.shape
    kk = embed.shape[0]
    xf = x.reshape(-1, d)
    dist, idx = _dist_argmax(xf, (embed + embed).T)
    quantize = _make_sc_gather(xf.shape[0], d)(idx, embed)
    return (quantize.reshape(b, s, d), idx.reshape(b, s),
            dist.reshape(b, s, kk))
